# Initial kernel scaffold; baseline (speedup 1.0000x reference)
import jax
import jax.numpy as jnp
from jax.experimental import pallas as pl


def kernel(x, edge_index, Wp, bp, Wb, bb, emb, Wt, bt, Wg1, bg1, Wg2, bg2, M1, mb1, g1, be1, M2, mb2, g2, be2, M3, mb3):
    raise NotImplementedError("write your pallas kernel here")


# speedup vs baseline: 2.0210x; 2.0210x over previous
import functools

import jax
import jax.numpy as jnp
from jax import lax
from jax.experimental import pallas as pl
from jax.experimental.pallas import tpu as pltpu
from jax.experimental.pallas import tpu_sc as plsc

N = 10000
E = 320000
H = 128

NC = 2
NS = 16
NW = NC * NS
K = 128
CH = 80
E_PAD = NW * CH * K
NP = 3
UR = 4000
AC = 4096
RPT = AC // NS


def _scat_body(hs_hbm, src_hbm, dst_hbm, out_hbm,
               idx_s, idx_d, rows0, rows1, zbuf, shared, gs0, gs1):
    c = lax.axis_index("c")
    s = lax.axis_index("s")
    w = c * NS + s
    pltpu.sync_copy(src_hbm.at[w], idx_s)

    @pl.loop(0, RPT)
    def _(i):
        for l in range(H // 16):
            zbuf[i, pl.ds(16 * l, 16)] = jnp.zeros((16,), jnp.int32)

    for p in range(NP):
        pltpu.sync_copy(dst_hbm.at[p, w], idx_d)
        pltpu.sync_copy(zbuf, shared.at[pl.ds(s * RPT, RPT)])
        plsc.subcore_barrier()

        pltpu.async_copy(hs_hbm.at[idx_s.at[0]], rows0, gs0)
        pltpu.async_copy(hs_hbm.at[idx_s.at[1]], rows1, gs1)

        @pl.loop(0, CH, step=2)
        def _(j):
            pltpu.make_async_copy(hs_hbm.at[idx_s.at[j]], rows0, gs0).wait()
            pltpu.sync_copy(rows0, shared.at[idx_d.at[j]], add=True)

            @pl.when(j + 2 < CH)
            def _():
                pltpu.async_copy(hs_hbm.at[idx_s.at[j + 2]], rows0, gs0)

            pltpu.make_async_copy(hs_hbm.at[idx_s.at[j + 1]], rows1, gs1).wait()
            pltpu.sync_copy(rows1, shared.at[idx_d.at[j + 1]], add=True)

            @pl.when(j + 3 < CH)
            def _():
                pltpu.async_copy(hs_hbm.at[idx_s.at[j + 3]], rows1, gs1)

        plsc.subcore_barrier()
        pltpu.sync_copy(shared.at[pl.ds(s * RPT, RPT)],
                        out_hbm.at[c, p, pl.ds(s * RPT, RPT)])


@functools.cache
def _sc_scatter_fn():
    return pl.kernel(
        _scat_body,
        out_type=jax.ShapeDtypeStruct((NC, NP, AC, H), jnp.int32),
        mesh=plsc.VectorSubcoreMesh(core_axis_name="c", subcore_axis_name="s"),
        scratch_types=[
            pltpu.VMEM((CH, K), jnp.int32),
            pltpu.VMEM((CH, K), jnp.int32),
            pltpu.VMEM((K, H), jnp.int32),
            pltpu.VMEM((K, H), jnp.int32),
            pltpu.VMEM((RPT, H), jnp.int32),
            pltpu.VMEM_SHARED((AC, H), jnp.int32),
            pltpu.SemaphoreType.DMA,
            pltpu.SemaphoreType.DMA,
        ],
    )


def _rebuild(q):
    return jnp.concatenate([q[0, :UR], q[1, :UR], q[2, :N - 2 * UR]], axis=0)


def _embed_body(x_ref, emb_ref, wb_ref, bb_ref, wp_ref, bp_ref, wt1_ref,
                wt2_ref, bt_ref, h1_ref):
    x = x_ref[...]
    hx = ((x[:, 0:1] * wb_ref[...] + bb_ref[...])
          + (x[:, 1:2] * wp_ref[...] + bp_ref[...]))
    h1_ref[...] = (
        jnp.dot(hx, wt1_ref[...], preferred_element_type=jnp.float32)
        + jnp.dot(emb_ref[...], wt2_ref[...], preferred_element_type=jnp.float32)
        + bt_ref[...]
    )


def _tc_embed(x, emb, wb, bb, wp, bp, wt1, wt2, bt):
    return pl.pallas_call(
        _embed_body,
        out_shape=jax.ShapeDtypeStruct((N, H), jnp.float32),
    )(x, emb, wb, bb, wp, bp, wt1, wt2, bt)


def _quant_scale(hmax, maxcnt):
    raw = (2.0 ** 30) / (hmax * (maxcnt + 1.0) + 1e-30)
    sc = jnp.exp2(jnp.floor(jnp.log2(raw)))
    return jnp.where(hmax > 0, jnp.minimum(sc, 2.0 ** 40), 1.0)


def _scale_body(s_ref, h1_ref, hsq_ref, hs_ref, dinv_ref, scale_ref, maxc_ref):
    cnt = _rebuild(s_ref[0, :, :, 0:1] + s_ref[1, :, :, 0:1]).astype(jnp.float32)
    dinv = lax.rsqrt(cnt + 1.0)
    hs = h1_ref[...] * dinv
    maxc = jnp.max(cnt)
    hmax = jnp.max(jnp.abs(hs))
    scale = _quant_scale(hmax, maxc)
    hsq_ref[...] = jnp.round(hs * scale).astype(jnp.int32)
    hs_ref[...] = hs
    dinv_ref[...] = dinv
    scale_ref[...] = jnp.full((1, 1), 1.0, jnp.float32) * scale
    maxc_ref[...] = jnp.full((1, 1), 1.0, jnp.float32) * maxc


def _tc_scale(s_acc, h1):
    return pl.pallas_call(
        _scale_body,
        out_shape=(
            jax.ShapeDtypeStruct((N, H), jnp.int32),
            jax.ShapeDtypeStruct((N, H), jnp.float32),
            jax.ShapeDtypeStruct((N, 1), jnp.float32),
            jax.ShapeDtypeStruct((1, 1), jnp.float32),
            jax.ShapeDtypeStruct((1, 1), jnp.float32),
        ),
    )(s_acc, h1)


def _gcn_body(s_ref, hs_ref, dinv_ref, scale_ref, maxc_ref, w_ref, b_ref,
              h_ref, hsnq_ref, hsn_ref, scalen_ref):
    dinv = dinv_ref[...]
    inv_scale = 1.0 / scale_ref[0, 0]
    ssum = _rebuild(s_ref[0] + s_ref[1]).astype(jnp.float32) * inv_scale
    agg = (ssum + hs_ref[...]) * dinv
    h = jnp.maximum(
        jnp.dot(agg, w_ref[...], preferred_element_type=jnp.float32) + b_ref[...],
        0.0,
    )
    hsn = h * dinv
    hmax = jnp.max(jnp.abs(hsn))
    scale = _quant_scale(hmax, maxc_ref[0, 0])
    h_ref[...] = h
    hsnq_ref[...] = jnp.round(hsn * scale).astype(jnp.int32)
    hsn_ref[...] = hsn
    scalen_ref[...] = jnp.full((1, 1), 1.0, jnp.float32) * scale


def _tc_gcn(s_acc, hs, dinv, scale, maxc, w, b):
    return pl.pallas_call(
        _gcn_body,
        out_shape=(
            jax.ShapeDtypeStruct((N, H), jnp.float32),
            jax.ShapeDtypeStruct((N, H), jnp.int32),
            jax.ShapeDtypeStruct((N, H), jnp.float32),
            jax.ShapeDtypeStruct((1, 1), jnp.float32),
        ),
    )(s_acc, hs, dinv, scale, maxc, w, b)


def _mlp_body(h3_ref, m1_ref, mb1_ref, g1_ref, be1_ref, m2_ref, mb2_ref,
              g2_ref, be2_ref, m3r_ref, mb3_ref, out_ref):
    eps = 1e-5
    y1 = jnp.dot(h3_ref[...], m1_ref[...], preferred_element_type=jnp.float32) + mb1_ref[...]
    mu1 = jnp.mean(y1, axis=0, keepdims=True)
    v1 = jnp.mean((y1 - mu1) * (y1 - mu1), axis=0, keepdims=True)
    z1 = jnp.maximum((y1 - mu1) * lax.rsqrt(v1 + eps) * g1_ref[...] + be1_ref[...], 0.0)
    y2 = jnp.dot(z1, m2_ref[...], preferred_element_type=jnp.float32) + mb2_ref[...]
    mu2 = jnp.mean(y2, axis=0, keepdims=True)
    v2 = jnp.mean((y2 - mu2) * (y2 - mu2), axis=0, keepdims=True)
    z2 = jnp.maximum((y2 - mu2) * lax.rsqrt(v2 + eps) * g2_ref[...] + be2_ref[...], 0.0)
    out_ref[...] = jnp.dot(z2, m3r_ref[...], preferred_element_type=jnp.float32) + mb3_ref[...]


def _tc_mlp(h3, m1, mb1, g1, be1, m2, mb2, g2, be2, m3r, mb3):
    return pl.pallas_call(
        _mlp_body,
        out_shape=jax.ShapeDtypeStruct((N, 1), jnp.float32),
    )(h3, m1, mb1, g1, be1, m2, mb2, g2, be2, m3r, mb3)


def kernel(x, edge_index, Wp, bp, Wb, bb, emb, Wt, bt, Wg1, bg1, Wg2, bg2,
           M1, mb1, g1, be1, M2, mb2, g2, be2, M3, mb3):
    src = edge_index[0]
    dst = edge_index[1]
    npad = E_PAD - E
    src3 = jnp.concatenate([src, jnp.zeros((npad,), jnp.int32)]).reshape(NW, CH, K)
    dstp = jnp.concatenate([dst, N + (jnp.arange(npad, dtype=jnp.int32) % (NP * UR - N))])
    passes = []
    for p in range(NP):
        local = dstp - p * UR
        ok = (local >= 0) & (local < UR)
        passes.append(jnp.where(ok, local, UR + (dstp & 63)))
    dst3 = jnp.stack(passes).reshape(NP, NW, CH, K)

    wt1, wt2 = Wt[:H], Wt[H:]

    h1 = _tc_embed(x, emb, Wb, bb[None, :], Wp, bp[None, :], wt1, wt2,
                   bt[None, :])

    wg = jnp.stack([Wg1, Wg1, Wg2])
    bg = jnp.stack([bg1[None, :], bg1[None, :], bg2[None, :]])

    def step(carry, xs):
        hsq, hs, dinv, scale, maxc, h = carry
        i, w, b = xs
        s_acc = _sc_scatter_fn()(hsq, src3, dst3)

        def branch_deg(_):
            hsq1, hs1, dinv1, scale1, maxc1 = _tc_scale(s_acc, h1)
            return hsq1, hs1, dinv1, scale1, maxc1, h

        def branch_gcn(_):
            hn, hsnq, hsn, scalen = _tc_gcn(s_acc, hs, dinv, scale, maxc, w, b)
            return hsnq, hsn, dinv, scalen, maxc, hn

        return lax.cond(i == 0, branch_deg, branch_gcn, None), None

    init = (jnp.ones((N, H), jnp.int32),
            jnp.ones((N, H), jnp.float32),
            jnp.zeros((N, 1), jnp.float32),
            jnp.ones((1, 1), jnp.float32),
            jnp.zeros((1, 1), jnp.float32),
            jnp.zeros((N, H), jnp.float32))
    (_, _, _, _, _, h3), _ = lax.scan(step, init, (jnp.arange(3), wg, bg))

    return _tc_mlp(h3, M1, mb1[None, :], g1[None, :], be1[None, :],
                   M2, mb2[None, :], g2[None, :], be2[None, :],
                   M3, mb3[None, :])
